# Initial kernel scaffold; baseline (speedup 1.0000x reference)
import jax
import jax.numpy as jnp
from jax.experimental import pallas as pl


def kernel(x):
    raise NotImplementedError("write your pallas kernel here")


# speedup vs baseline: 18.8893x; 18.8893x over previous
import functools

import jax
import jax.numpy as jnp
import numpy as np
from jax import lax
from jax.experimental import pallas as pl
from jax.experimental.pallas import tpu as pltpu

_INTMIN = np.int32(-(2**31))


def _median_body(x_ref, o_ref, ks_ref, *, rank):
    i = pl.program_id(1)
    m = lax.bitcast_convert_type(x_ref[0], jnp.int32)
    ks_ref[...] = jnp.where(m < 0, ~m ^ _INTMIN, m)
    cb = x_ref.shape[2]

    def step(_, carry):
        p, bitv = carry
        trial = p | bitv
        cnt = jnp.sum(
            (ks_ref[...] < (trial ^ _INTMIN)).astype(jnp.int32),
            axis=0,
            keepdims=True,
        )
        p = jnp.where(cnt <= rank, trial, p)
        return p, lax.shift_right_logical(bitv, 1)

    p0 = jnp.zeros((1, cb), jnp.int32)
    p, _ = lax.fori_loop(0, 32, step, (p0, _INTMIN))
    m_out = jnp.where(p < 0, p ^ _INTMIN, ~p)
    o_ref[pl.ds(i, 1), :] = lax.bitcast_convert_type(m_out, jnp.float32)


def kernel(x):
    b, n, c = x.shape
    rank = (n - 1) // 2
    cb = 256
    grid = (c // cb, b)
    return pl.pallas_call(
        functools.partial(_median_body, rank=rank),
        grid=grid,
        in_specs=[
            pl.BlockSpec((1, n, cb), lambda j, i: (i, 0, j)),
        ],
        out_specs=pl.BlockSpec((b, cb), lambda j, i: (0, j)),
        out_shape=jax.ShapeDtypeStruct((b, c), jnp.float32),
        scratch_shapes=[pltpu.VMEM((n, cb), jnp.int32)],
    )(x)
